# baseline (speedup 1.0000x reference)
import functools

import jax
import jax.numpy as jnp
from jax import lax
from jax.experimental import pallas as pl
from jax.experimental.pallas import tpu as pltpu
from jax.experimental.pallas import tpu_sc as plsc

B = 16
S = 16
V = 100000
L = 16
VCHUNKS = V // L


CH = 12800
NACC = 4
VALN = 99968
TAILN = V - VALN
_CHUNKS = [(k * CH, CH) for k in range(7)] + [(7 * CH, VALN - 7 * CH)]


def _argmax_rows_body(table_hbm, tail_hbm, seq_hbm, out_hbm, sl_v, buf_a,
                      buf_b, tail_v, outv, sem_a, sem_b, sem_t):
    c = lax.axis_index("c")
    s = lax.axis_index("s")
    b = c * 8 + s

    @pl.when(s < 8)
    def _():
        pltpu.sync_copy(seq_hbm, sl_v)
        iota = lax.iota(jnp.int32, L)
        slb = plsc.load_gather(sl_v.at[:], [jnp.full((L,), b, jnp.int32)])
        r = slb[0] - 1

        bufs = [buf_a, buf_b]
        sems = [sem_a, sem_b]

        def start(k):
            off, ln = _CHUNKS[k]
            return pltpu.async_copy(
                table_hbm.at[b, r, pl.ds(off, ln)],
                bufs[k % 2].at[pl.ds(0, ln)], sems[k % 2])

        def body_for(buf):
            def body(i, carry):
                cms, cis, base = carry
                cms, cis = list(cms), list(cis)
                for j in range(NACC):
                    v = buf[pl.ds(i * (NACC * L) + j * L, L)]
                    m = v > cms[j]
                    cms[j] = jnp.where(m, v, cms[j])
                    cis[j] = jnp.where(m, base + j * L, cis[j])
                return tuple(cms), tuple(cis), base + NACC * L
            return body

        cms = (jnp.full((L,), -jnp.inf, jnp.float32),) * NACC
        cis = (jnp.zeros((L,), jnp.int32),) * NACC
        cp = start(0)
        cp_t = pltpu.async_copy(tail_hbm.at[b, r], tail_v, sem_t)
        for k, (off, ln) in enumerate(_CHUNKS):
            cp.wait()
            if k + 1 < len(_CHUNKS):
                cp = start(k + 1)
            iters = ln // (NACC * L)
            unroll = 8 if iters % 8 == 0 else 6
            cms, cis, _ = lax.fori_loop(
                0, iters, body_for(bufs[k % 2]), (cms, cis, iota + off),
                unroll=unroll)
        cp_t.wait()
        cms, cis = list(cms), list(cis)
        for j in range(TAILN // L):
            v = tail_v[pl.ds(j * L, L)]
            base = iota + (VALN + j * L)
            m = v > cms[j]
            cms[j] = jnp.where(m, v, cms[j])
            cis[j] = jnp.where(m, base, cis[j])

        def merge(a, b):
            (cma, cia), (cmb, cib) = a, b
            better = (cmb > cma) | ((cmb == cma) & (cib < cia))
            return (jnp.where(better, cmb, cma),
                    jnp.where(better, cib, cia))

        pairs = [(cms[j], cis[j]) for j in range(NACC)]
        while len(pairs) > 1:
            pairs = [merge(pairs[i], pairs[i + 1])
                     for i in range(0, len(pairs), 2)]
        cm, ci = pairs[0]

        for shift in (8, 4, 2, 1):
            perm = iota ^ shift
            om = cm.at[perm].get(mode="promise_in_bounds")
            oi = ci.at[perm].get(mode="promise_in_bounds")
            better = (om > cm) | ((om == cm) & (oi < ci))
            cm = jnp.where(better, om, cm)
            ci = jnp.where(better, oi, ci)
        outv[...] = ci
        pltpu.sync_copy(outv, out_hbm.at[b])


def kernel(logits, seq_lens):
    table = logits
    tail = lax.slice(logits, (0, 0, VALN), (B, S, V))
    sl = seq_lens.astype(jnp.int32)
    mesh = plsc.VectorSubcoreMesh(core_axis_name="c", subcore_axis_name="s")
    run = functools.partial(
        pl.kernel,
        mesh=mesh,
        out_type=jax.ShapeDtypeStruct((B, L), jnp.int32),
        scratch_types=[
            pltpu.VMEM((L,), jnp.int32),
            pltpu.VMEM((CH,), jnp.float32),
            pltpu.VMEM((CH,), jnp.float32),
            pltpu.VMEM((TAILN,), jnp.float32),
            pltpu.VMEM((L,), jnp.int32),
            pltpu.SemaphoreType.DMA,
            pltpu.SemaphoreType.DMA,
            pltpu.SemaphoreType.DMA,
        ],
        compiler_params=pltpu.CompilerParams(needs_layout_passes=False),
    )(_argmax_rows_body)
    out = run(table, tail, sl)
    return out[:, :1]


# speedup vs baseline: 1.1368x; 1.1368x over previous
import functools

import jax
import jax.numpy as jnp
from jax import lax
from jax.experimental import pallas as pl
from jax.experimental.pallas import tpu as pltpu
from jax.experimental.pallas import tpu_sc as plsc

B = 16
S = 16
V = 100000
L = 16

VALN = 99968
TAILN = V - VALN
HALF = 51200
H1OFF = VALN - HALF
CH = 12800
NACC = 4


def _argmax_rows_body(table_hbm, tail_hbm, seq_hbm, out_hbm, sl_v, buf_a,
                      buf_b, tail_v, stage_m, stage_i, outv, sh_m, sh_i,
                      sem_a, sem_b, sem_t):
    c = lax.axis_index("c")
    s = lax.axis_index("s")
    h = s // 8
    b = c * 8 + (s % 8)

    pltpu.sync_copy(seq_hbm, sl_v)
    iota = lax.iota(jnp.int32, L)
    slb = plsc.load_gather(sl_v.at[:], [jnp.full((L,), b, jnp.int32)])
    r = slb[0] - 1
    off = h * H1OFF

    bufs = [buf_a, buf_b]
    sems = [sem_a, sem_b]

    def start(k):
        return pltpu.async_copy(
            table_hbm.at[b, r, pl.ds(off + k * CH, CH)],
            bufs[k % 2], sems[k % 2])

    def body_for(buf, base0):
        def body(i, carry):
            cms, cis, base = carry
            cms, cis = list(cms), list(cis)
            for j in range(NACC):
                v = buf[pl.ds(i * (NACC * L) + j * L, L)]
                m = v > cms[j]
                cms[j] = jnp.where(m, v, cms[j])
                cis[j] = jnp.where(m, base + j * L, cis[j])
            return tuple(cms), tuple(cis), base + NACC * L
        return body

    cms = (jnp.full((L,), -jnp.inf, jnp.float32),) * NACC
    cis = (jnp.zeros((L,), jnp.int32),) * NACC
    cp = start(0)
    cp_t = pltpu.async_copy(tail_hbm.at[b, r], tail_v, sem_t)
    nch = HALF // CH
    for k in range(nch):
        cp.wait()
        if k + 1 < nch:
            cp = start(k + 1)
        cms, cis, _ = lax.fori_loop(
            0, CH // (NACC * L), body_for(bufs[k % 2], off),
            (cms, cis, iota + off + k * CH), unroll=8)

    cp_t.wait()
    cms, cis = list(cms), list(cis)
    for j in range(TAILN // L):
        v = tail_v[pl.ds(j * L, L)]
        base = iota + (VALN + j * L)
        m = v > cms[j]
        cms[j] = jnp.where(m, v, cms[j])
        cis[j] = jnp.where(m, base, cis[j])

    def merge(a, bb):
        (cma, cia), (cmb, cib) = a, bb
        better = (cmb > cma) | ((cmb == cma) & (cib < cia))
        return (jnp.where(better, cmb, cma), jnp.where(better, cib, cia))

    pairs = [(cms[j], cis[j]) for j in range(NACC)]
    while len(pairs) > 1:
        pairs = [merge(pairs[i], pairs[i + 1])
                 for i in range(0, len(pairs), 2)]
    cm, ci = pairs[0]

    stage_m[...] = cm
    stage_i[...] = ci
    pltpu.sync_copy(stage_m, sh_m.at[s])
    pltpu.sync_copy(stage_i, sh_i.at[s])
    plsc.subcore_barrier()

    @pl.when(s < 8)
    def _():
        pltpu.sync_copy(sh_m.at[s + 8], stage_m)
        pltpu.sync_copy(sh_i.at[s + 8], stage_i)
        om = stage_m[...]
        oi = stage_i[...]
        cm2, ci2 = merge((cm, ci), (om, oi))

        for shift in (8, 4, 2, 1):
            perm = iota ^ shift
            om = cm2.at[perm].get(mode="promise_in_bounds")
            oi = ci2.at[perm].get(mode="promise_in_bounds")
            cm2, ci2 = merge((cm2, ci2), (om, oi))
        outv[...] = ci2
        pltpu.sync_copy(outv, out_hbm.at[b])


def kernel(logits, seq_lens):
    tail = lax.slice(logits, (0, 0, VALN), (B, S, V))
    sl = seq_lens.astype(jnp.int32)
    mesh = plsc.VectorSubcoreMesh(core_axis_name="c", subcore_axis_name="s")
    run = functools.partial(
        pl.kernel,
        mesh=mesh,
        out_type=jax.ShapeDtypeStruct((B, L), jnp.int32),
        scratch_types=[
            pltpu.VMEM((L,), jnp.int32),
            pltpu.VMEM((CH,), jnp.float32),
            pltpu.VMEM((CH,), jnp.float32),
            pltpu.VMEM((TAILN,), jnp.float32),
            pltpu.VMEM((L,), jnp.float32),
            pltpu.VMEM((L,), jnp.int32),
            pltpu.VMEM((L,), jnp.int32),
            pltpu.VMEM_SHARED((16, L), jnp.float32),
            pltpu.VMEM_SHARED((16, L), jnp.int32),
            pltpu.SemaphoreType.DMA,
            pltpu.SemaphoreType.DMA,
            pltpu.SemaphoreType.DMA,
        ],
        compiler_params=pltpu.CompilerParams(needs_layout_passes=False),
    )(_argmax_rows_body)
    out = run(logits, tail, sl)
    return out[:, :1]
